# Initial kernel scaffold; baseline (speedup 1.0000x reference)
import jax
import jax.numpy as jnp
from jax.experimental import pallas as pl


def kernel(self_tensor, index, src):
    raise NotImplementedError("write your pallas kernel here")


# speedup vs baseline: 3.6034x; 3.6034x over previous
import functools

import jax
import jax.numpy as jnp
from jax import lax
from jax.experimental import pallas as pl
from jax.experimental.pallas import tpu as pltpu
from jax.experimental.pallas import tpu_sc as plsc

_LANES = 16
_NC = 2
_NS = 16
_NW = _NC * _NS
_CH = 4096


def _copy_body(a_ref, o_ref):
    o_ref[...] = a_ref[...]


def _xpose_body(idx_ref, src_ref, idxt_ref, srct_ref):
    idxt_ref[...] = idx_ref[...].T
    srct_ref[...] = src_ref[...].T


@functools.cache
def _make_sc_scatter(m, b, d):
    cpw = d // _NW
    n_chunks = b // _CH
    vregs = _CH // _LANES
    mesh = plsc.VectorSubcoreMesh(core_axis_name="c", subcore_axis_name="s")

    @functools.partial(
        pl.kernel,
        mesh=mesh,
        compiler_params=pltpu.CompilerParams(needs_layout_passes=False),
        scratch_types=[
            pltpu.VMEM((m,), jnp.int32),
            pltpu.VMEM((b,), jnp.float32),
            pltpu.VMEM((_CH,), jnp.int32),
            pltpu.VMEM((_CH,), jnp.int32),
            pltpu.VMEM((_CH,), jnp.float32),
            pltpu.SemaphoreType.DMA,
        ],
    )
    def sc_scatter(out_ref, idxt_hbm, srct_hbm,
                   win_ref, srcc_ref, idx_ref, off_ref, val_ref, sem):
        cc = lax.axis_index("c")
        ss = lax.axis_index("s")
        wid = ss * _NC + cc
        lanes = lax.iota(jnp.int32, _LANES)
        for k in range(cpw):
            j = wid * cpw + k
            pltpu.sync_copy(srct_hbm.at[j], srcc_ref)

            def p1_chunk(ci, _, j=j):
                pltpu.sync_copy(idxt_hbm.at[j, pl.ds(ci * _CH, _CH)], idx_ref)
                base = ci * _CH

                def p1_vreg(v, _):
                    t = idx_ref[pl.ds(v * _LANES, _LANES)]
                    ivec = base + v * _LANES + lanes
                    plsc.store_scatter(win_ref, [t], ivec)
                    w = plsc.load_gather(win_ref, [t])

                    def fix_cond(w_):
                        return jnp.any(w_ < ivec)

                    def fix_body(w_):
                        plsc.store_scatter(win_ref, [t], ivec, mask=w_ < ivec)
                        return plsc.load_gather(win_ref, [t])

                    lax.while_loop(fix_cond, fix_body, w)
                    return 0

                lax.fori_loop(0, vregs, p1_vreg, 0)
                return 0

            lax.fori_loop(0, n_chunks, p1_chunk, 0)

            def p2_chunk(ci, _, j=j):
                pltpu.sync_copy(idxt_hbm.at[j, pl.ds(ci * _CH, _CH)], idx_ref)

                def p2_vreg(v, _):
                    sl = pl.ds(v * _LANES, _LANES)
                    t = idx_ref[sl]
                    w = plsc.load_gather(win_ref, [t])
                    vals = plsc.load_gather(srcc_ref, [w])
                    off_ref[sl] = t * d + j
                    val_ref[sl] = vals
                    return 0

                lax.fori_loop(0, vregs, p2_vreg, 0)
                pltpu.async_copy(val_ref, out_ref.at[off_ref], sem).wait()
                return 0

            lax.fori_loop(0, n_chunks, p2_chunk, 0)

    return sc_scatter


def kernel(self_tensor, index, src):
    m, d = self_tensor.shape
    b = index.shape[0]
    n = m * d
    copy_grid = 25
    xpose_grid = 8

    a_flat = self_tensor.reshape(n)
    out0 = pl.pallas_call(
        _copy_body,
        grid=(copy_grid,),
        in_specs=[pl.BlockSpec((n // copy_grid,), lambda g: (g,))],
        out_specs=pl.BlockSpec((n // copy_grid,), lambda g: (g,)),
        out_shape=jax.ShapeDtypeStruct((n,), jnp.float32),
    )(a_flat)

    idxt, srct = pl.pallas_call(
        _xpose_body,
        grid=(xpose_grid,),
        in_specs=[
            pl.BlockSpec((b // xpose_grid, d), lambda g: (g, 0)),
            pl.BlockSpec((b // xpose_grid, d), lambda g: (g, 0)),
        ],
        out_specs=[
            pl.BlockSpec((d, b // xpose_grid), lambda g: (0, g)),
            pl.BlockSpec((d, b // xpose_grid), lambda g: (0, g)),
        ],
        out_shape=[
            jax.ShapeDtypeStruct((d, b), jnp.int32),
            jax.ShapeDtypeStruct((d, b), jnp.float32),
        ],
    )(index, src)

    out_ref = jax.new_ref(out0)
    _make_sc_scatter(m, b, d)(out_ref, idxt, srct)
    return jax.freeze(out_ref).reshape(m, d)


# speedup vs baseline: 3.6418x; 1.0106x over previous
import functools

import jax
import jax.numpy as jnp
from jax import lax
from jax.experimental import pallas as pl
from jax.experimental.pallas import tpu as pltpu
from jax.experimental.pallas import tpu_sc as plsc

_LANES = 16
_NC = 2
_NS = 16
_NW = _NC * _NS
_CH = 4096
_G1 = 16
_SCH = 1024
_G2 = 8


def _copy_body(a_ref, o_ref):
    o_ref[...] = a_ref[...]


def _xpose_body(idx_ref, src_ref, idxt_ref, srct_ref):
    idxt_ref[...] = idx_ref[...].T
    srct_ref[...] = src_ref[...].T


@functools.cache
def _make_sc_scatter(m, b, d):
    cpw = d // _NW
    n_chunks = b // _CH
    g1_iters = _CH // (_LANES * _G1)
    n_sub = _CH // _SCH
    g2_iters = _SCH // (_LANES * _G2)
    mesh = plsc.VectorSubcoreMesh(core_axis_name="c", subcore_axis_name="s")

    @functools.partial(
        pl.kernel,
        mesh=mesh,
        compiler_params=pltpu.CompilerParams(needs_layout_passes=False),
        scratch_types=[
            pltpu.VMEM((m,), jnp.int32),
            pltpu.VMEM((b,), jnp.float32),
            pltpu.VMEM((_CH,), jnp.int32),
            pltpu.VMEM((_CH,), jnp.int32),
            pltpu.VMEM((_SCH,), jnp.int32),
            pltpu.VMEM((_SCH,), jnp.int32),
            pltpu.VMEM((_SCH,), jnp.float32),
            pltpu.VMEM((_SCH,), jnp.float32),
            pltpu.SemaphoreType.DMA,
            pltpu.SemaphoreType.DMA,
            pltpu.SemaphoreType.DMA,
            pltpu.SemaphoreType.DMA,
        ],
    )
    def sc_scatter(out_ref, idxt_hbm, srct_hbm,
                   win_ref, srcc_ref, idx_a, idx_b, off_a, off_b, val_a, val_b,
                   sem_i0, sem_i1, sem_s0, sem_s1):
        idx_bufs = (idx_a, idx_b)
        off_bufs = (off_a, off_b)
        val_bufs = (val_a, val_b)
        sem_i = (sem_i0, sem_i1)
        sem_s = (sem_s0, sem_s1)
        cc = lax.axis_index("c")
        ss = lax.axis_index("s")
        wid = ss * _NC + cc
        lanes = lax.iota(jnp.int32, _LANES)

        def column(col, _):
            j = wid * cpw + col
            pltpu.sync_copy(srct_hbm.at[j], srcc_ref)

            def load_idx(c):
                return pltpu.async_copy(
                    idxt_hbm.at[j, pl.ds(c * _CH, _CH)],
                    idx_bufs[c % 2], sem_i[c % 2])

            idesc = load_idx(0)
            for c in range(n_chunks):
                nxt = load_idx(c + 1) if c + 1 < n_chunks else None
                idesc.wait()
                cb = idx_bufs[c % 2]

                def p1_group(gi, _, c=c, cb=cb):
                    base = c * _CH + gi * (_LANES * _G1)
                    ts = []
                    ivs = []
                    for v in range(_G1):
                        t = cb[pl.ds(gi * (_LANES * _G1) + v * _LANES, _LANES)]
                        iv = base + v * _LANES + lanes
                        plsc.store_scatter(win_ref, [t], iv)
                        ts.append(t)
                        ivs.append(iv)
                    bad = None
                    for v in range(_G1):
                        w = plsc.load_gather(win_ref, [ts[v]])
                        mv = w < ivs[v]
                        bad = mv if bad is None else (bad | mv)

                    def fix_cond(nbad):
                        return nbad > 0

                    def fix_body(nbad):
                        accv = jnp.zeros((_LANES,), jnp.int32)
                        for v in range(_G1):
                            w = plsc.load_gather(win_ref, [ts[v]])
                            mv = w < ivs[v]
                            plsc.store_scatter(win_ref, [ts[v]], ivs[v], mask=mv)
                        for v in range(_G1):
                            w = plsc.load_gather(win_ref, [ts[v]])
                            accv = accv + (w < ivs[v]).astype(jnp.int32)
                        return jnp.sum(accv)

                    lax.while_loop(fix_cond, fix_body,
                                   jnp.any(bad).astype(jnp.int32))
                    return 0

                lax.fori_loop(0, g1_iters, p1_group, 0)
                idesc = nxt

            sdescs = [None, None]
            idesc = load_idx(0)
            for c in range(n_chunks):
                nxt = load_idx(c + 1) if c + 1 < n_chunks else None
                idesc.wait()
                cb = idx_bufs[c % 2]
                for sub in range(n_sub):
                    sb = (c * n_sub + sub) % 2
                    if sdescs[sb] is not None:
                        sdescs[sb].wait()
                    ob, vb = off_bufs[sb], val_bufs[sb]

                    def p2_group(gi, _, cb=cb, sub=sub, ob=ob, vb=vb):
                        for v in range(_G2):
                            sl_in = pl.ds(sub * _SCH
                                          + gi * (_LANES * _G2) + v * _LANES,
                                          _LANES)
                            sl_out = pl.ds(gi * (_LANES * _G2) + v * _LANES,
                                           _LANES)
                            t = cb[sl_in]
                            w = plsc.load_gather(win_ref, [t])
                            vals = plsc.load_gather(srcc_ref, [w])
                            ob[sl_out] = t * d + j
                            vb[sl_out] = vals
                        return 0

                    lax.fori_loop(0, g2_iters, p2_group, 0)
                    sdescs[sb] = pltpu.async_copy(
                        vb, out_ref.at[ob], sem_s[sb])
                idesc = nxt
            for sd in sdescs:
                if sd is not None:
                    sd.wait()
            return 0

        lax.fori_loop(0, cpw, column, 0)

    return sc_scatter


def kernel(self_tensor, index, src):
    m, d = self_tensor.shape
    b = index.shape[0]
    n = m * d
    copy_grid = 25
    xpose_grid = 8

    a_flat = self_tensor.reshape(n)
    out0 = pl.pallas_call(
        _copy_body,
        grid=(copy_grid,),
        in_specs=[pl.BlockSpec((n // copy_grid,), lambda g: (g,))],
        out_specs=pl.BlockSpec((n // copy_grid,), lambda g: (g,)),
        out_shape=jax.ShapeDtypeStruct((n,), jnp.float32),
    )(a_flat)

    idxt, srct = pl.pallas_call(
        _xpose_body,
        grid=(xpose_grid,),
        in_specs=[
            pl.BlockSpec((b // xpose_grid, d), lambda g: (g, 0)),
            pl.BlockSpec((b // xpose_grid, d), lambda g: (g, 0)),
        ],
        out_specs=[
            pl.BlockSpec((d, b // xpose_grid), lambda g: (0, g)),
            pl.BlockSpec((d, b // xpose_grid), lambda g: (0, g)),
        ],
        out_shape=[
            jax.ShapeDtypeStruct((d, b), jnp.int32),
            jax.ShapeDtypeStruct((d, b), jnp.float32),
        ],
    )(index, src)

    out_ref = jax.new_ref(out0)
    _make_sc_scatter(m, b, d)(out_ref, idxt, srct)
    return jax.freeze(out_ref).reshape(m, d)


# speedup vs baseline: 3.6418x; 1.0000x over previous
import functools

import jax
import jax.numpy as jnp
from jax import lax
from jax.experimental import pallas as pl
from jax.experimental.pallas import tpu as pltpu
from jax.experimental.pallas import tpu_sc as plsc

_LANES = 16
_NC = 2
_NS = 16
_NW = _NC * _NS
_CH = 4096
_G1 = 16
_SCH = 1024
_G2 = 8


def _copy_body(a_ref, o_ref):
    o_ref[...] = a_ref[...]


def _xpose_body(idx_ref, src_ref, idxt_ref, srct_ref):
    idxt_ref[...] = idx_ref[...].T
    srct_ref[...] = src_ref[...].T


@functools.cache
def _make_sc_scatter(m, b, d):
    cpw = d // _NW
    n_chunks = b // _CH
    g1_iters = _CH // (_LANES * _G1)
    n_sub = _CH // _SCH
    g2_iters = _SCH // (_LANES * _G2)
    mesh = plsc.VectorSubcoreMesh(core_axis_name="c", subcore_axis_name="s")

    @functools.partial(
        pl.kernel,
        mesh=mesh,
        compiler_params=pltpu.CompilerParams(needs_layout_passes=False),
        scratch_types=[
            pltpu.VMEM((m,), jnp.int32),
            pltpu.VMEM((b,), jnp.float32),
            pltpu.VMEM((_CH,), jnp.int32),
            pltpu.VMEM((_CH,), jnp.int32),
            pltpu.VMEM((_SCH,), jnp.int32),
            pltpu.VMEM((_SCH,), jnp.int32),
            pltpu.VMEM((_SCH,), jnp.float32),
            pltpu.VMEM((_SCH,), jnp.float32),
            pltpu.SemaphoreType.DMA,
            pltpu.SemaphoreType.DMA,
            pltpu.SemaphoreType.DMA,
            pltpu.SemaphoreType.DMA,
        ],
    )
    def sc_scatter(out_ref, idxt_hbm, srct_hbm,
                   win_ref, srcc_ref, idx_a, idx_b, off_a, off_b, val_a, val_b,
                   sem_i0, sem_i1, sem_s0, sem_s1):
        idx_bufs = (idx_a, idx_b)
        off_bufs = (off_a, off_b)
        val_bufs = (val_a, val_b)
        sem_i = (sem_i0, sem_i1)
        sem_s = (sem_s0, sem_s1)
        cc = lax.axis_index("c")
        ss = lax.axis_index("s")
        wid = ss * _NC + cc
        lanes = lax.iota(jnp.int32, _LANES)

        def column(col, _):
            j = wid * cpw + col
            with jax.named_scope("src_col_load"):
                pltpu.sync_copy(srct_hbm.at[j], srcc_ref)

            def load_idx(c):
                return pltpu.async_copy(
                    idxt_hbm.at[j, pl.ds(c * _CH, _CH)],
                    idx_bufs[c % 2], sem_i[c % 2])

            ns1 = jax.named_scope("pass1")
            ns1.__enter__()
            idesc = load_idx(0)
            for c in range(n_chunks):
                nxt = load_idx(c + 1) if c + 1 < n_chunks else None
                idesc.wait()
                cb = idx_bufs[c % 2]

                def p1_group(gi, _, c=c, cb=cb):
                    base = c * _CH + gi * (_LANES * _G1)
                    ts = []
                    ivs = []
                    for v in range(_G1):
                        t = cb[pl.ds(gi * (_LANES * _G1) + v * _LANES, _LANES)]
                        iv = base + v * _LANES + lanes
                        plsc.store_scatter(win_ref, [t], iv)
                        ts.append(t)
                        ivs.append(iv)
                    bad = None
                    for v in range(_G1):
                        w = plsc.load_gather(win_ref, [ts[v]])
                        mv = w < ivs[v]
                        bad = mv if bad is None else (bad | mv)

                    def fix_cond(nbad):
                        return nbad > 0

                    def fix_body(nbad):
                        accv = jnp.zeros((_LANES,), jnp.int32)
                        for v in range(_G1):
                            w = plsc.load_gather(win_ref, [ts[v]])
                            mv = w < ivs[v]
                            plsc.store_scatter(win_ref, [ts[v]], ivs[v], mask=mv)
                        for v in range(_G1):
                            w = plsc.load_gather(win_ref, [ts[v]])
                            accv = accv + (w < ivs[v]).astype(jnp.int32)
                        return jnp.sum(accv)

                    lax.while_loop(fix_cond, fix_body,
                                   jnp.any(bad).astype(jnp.int32))
                    return 0

                lax.fori_loop(0, g1_iters, p1_group, 0)
                idesc = nxt

            ns1.__exit__(None, None, None)
            ns2 = jax.named_scope("pass2")
            ns2.__enter__()
            sdescs = [None, None]
            idesc = load_idx(0)
            for c in range(n_chunks):
                nxt = load_idx(c + 1) if c + 1 < n_chunks else None
                idesc.wait()
                cb = idx_bufs[c % 2]
                for sub in range(n_sub):
                    sb = (c * n_sub + sub) % 2
                    if sdescs[sb] is not None:
                        sdescs[sb].wait()
                    ob, vb = off_bufs[sb], val_bufs[sb]

                    def p2_group(gi, _, cb=cb, sub=sub, ob=ob, vb=vb):
                        for v in range(_G2):
                            sl_in = pl.ds(sub * _SCH
                                          + gi * (_LANES * _G2) + v * _LANES,
                                          _LANES)
                            sl_out = pl.ds(gi * (_LANES * _G2) + v * _LANES,
                                           _LANES)
                            t = cb[sl_in]
                            w = plsc.load_gather(win_ref, [t])
                            vals = plsc.load_gather(srcc_ref, [w])
                            ob[sl_out] = t * d + j
                            vb[sl_out] = vals
                        return 0

                    lax.fori_loop(0, g2_iters, p2_group, 0)
                    sdescs[sb] = pltpu.async_copy(
                        vb, out_ref.at[ob], sem_s[sb])
                idesc = nxt
            for sd in sdescs:
                if sd is not None:
                    sd.wait()
            ns2.__exit__(None, None, None)
            return 0

        lax.fori_loop(0, cpw, column, 0)

    return sc_scatter


def kernel(self_tensor, index, src):
    m, d = self_tensor.shape
    b = index.shape[0]
    n = m * d
    copy_grid = 25
    xpose_grid = 8

    a_flat = self_tensor.reshape(n)
    out0 = pl.pallas_call(
        _copy_body,
        grid=(copy_grid,),
        in_specs=[pl.BlockSpec((n // copy_grid,), lambda g: (g,))],
        out_specs=pl.BlockSpec((n // copy_grid,), lambda g: (g,)),
        out_shape=jax.ShapeDtypeStruct((n,), jnp.float32),
    )(a_flat)

    idxt, srct = pl.pallas_call(
        _xpose_body,
        grid=(xpose_grid,),
        in_specs=[
            pl.BlockSpec((b // xpose_grid, d), lambda g: (g, 0)),
            pl.BlockSpec((b // xpose_grid, d), lambda g: (g, 0)),
        ],
        out_specs=[
            pl.BlockSpec((d, b // xpose_grid), lambda g: (0, g)),
            pl.BlockSpec((d, b // xpose_grid), lambda g: (0, g)),
        ],
        out_shape=[
            jax.ShapeDtypeStruct((d, b), jnp.int32),
            jax.ShapeDtypeStruct((d, b), jnp.float32),
        ],
    )(index, src)

    out_ref = jax.new_ref(out0)
    _make_sc_scatter(m, b, d)(out_ref, idxt, srct)
    return jax.freeze(out_ref).reshape(m, d)
